# baseline (speedup 1.0000x reference)
import functools

import jax
import jax.numpy as jnp
from jax import lax
from jax.experimental import pallas as pl
from jax.experimental.pallas import tpu as pltpu
from jax.experimental.pallas import tpu_sc as plsc

N = 10000
IN_DIM = 19
HID = 256
BOTT = 128
Q = 2
QW = HID // Q
NPAD = 10240
NS = 16
NC = 2
CHUNK = 128
TCH = 163840 // (16 * CHUNK)
HC = TCH // 2
PT = TCH * CHUNK
EPAD = PT * NS
MPAD = 1536
XPAD = 128
BT = 1000
LNEPS = 1e-5


def _build_agg():
    mesh = plsc.VectorSubcoreMesh(core_axis_name="c", subcore_axis_name="s",
                                  num_cores=NC, num_subcores=NS)

    @functools.partial(
        pl.kernel,
        out_type=jax.ShapeDtypeStruct((Q * NPAD, QW), jnp.float32),
        mesh=mesh,
        scratch_types=[
            pltpu.VMEM((HC, CHUNK), jnp.int32),
            pltpu.VMEM((HC, CHUNK), jnp.int32),
            pltpu.VMEM((CHUNK, QW), jnp.float32),
            pltpu.VMEM_SHARED((NPAD, QW), jnp.float32),
            pltpu.SemaphoreType.DMA,
        ],
    )
    def agg(h4, gidx, sidx, out, gv, sv, rows, acc, sem):
        cid = lax.axis_index("c")
        sid = lax.axis_index("s")
        stripe = NPAD // NS
        ZROWS = 128

        for p in range(Q // NC):
            qid = cid * (Q // NC) + p
            row = qid * NS + sid

            @pl.loop(0, ZROWS)
            def _zr(r):
                @pl.loop(0, QW // 16)
                def _zc(k):
                    rows[r, pl.ds(k * 16, 16)] = jnp.zeros((16,),
                                                           jnp.float32)

            @pl.loop(0, stripe // ZROWS)
            def _za(k):
                pltpu.sync_copy(rows.at[pl.ds(0, ZROWS)],
                                acc.at[pl.ds(sid * stripe + k * ZROWS,
                                             ZROWS)])

            plsc.subcore_barrier()

            for half in range(2):
                pltpu.sync_copy(gidx.at[row, half], gv)
                pltpu.sync_copy(sidx.at[sid, half], sv)

                @pl.loop(0, HC)
                def _edges(j):
                    pltpu.async_copy(h4.at[gv.at[j]], rows, sem).wait()
                    pltpu.sync_copy(rows, acc.at[sv.at[j]], add=True)

            plsc.subcore_barrier()
            pltpu.sync_copy(acc.at[pl.ds(sid * stripe, stripe)],
                            out.at[pl.ds(qid * NPAD + sid * stripe, stripe)])
            plsc.subcore_barrier()

    return agg


_agg_cache = []


def _agg(h4, gidx, sidx):
    if not _agg_cache:
        _agg_cache.append(_build_agg())
    return _agg_cache[0](h4, gidx, sidx)


def _ln(h, g, b):
    m = jnp.mean(h, axis=-1, keepdims=True)
    v = jnp.mean((h - m) ** 2, axis=-1, keepdims=True)
    return (h - m) / jnp.sqrt(v + LNEPS) * g + b


def _cat(ref):
    return jnp.concatenate([ref[i] for i in range(Q)], axis=1)


def _split_out(o_ref, v):
    for i in range(Q):
        o_ref[i] = v[:, i * QW:(i + 1) * QW]


def _mask_flag(m_ref, i):
    rows = lax.broadcasted_iota(jnp.int32, (BT, 1), 0) + i * BT
    flag = jnp.any(m_ref[0][None, :] == rows, axis=1, keepdims=True)
    return flag.astype(jnp.float32)


def _enc_in_body(x_ref, m_ref, w1_ref, w1f_ref, g_ref, b_ref, w2_ref, o_ref):
    flag = _mask_flag(m_ref, pl.program_id(0))
    xm = x_ref[...] * (1.0 - flag)
    z = jnp.dot(xm, w1_ref[...], preferred_element_type=jnp.float32,
                 precision=lax.Precision.HIGHEST)
    z = z + flag * w1f_ref[...]
    z = jnp.maximum(_ln(z, g_ref[...], b_ref[...]), 0.0)
    h = jnp.dot(z, w2_ref[...], preferred_element_type=jnp.float32,
                 precision=lax.Precision.HIGHEST)
    _split_out(o_ref, h)


def _dec_in_body(x_ref, m_ref, h_ref, w1_ref, w1f_ref, w1h_ref, g_ref, b_ref,
                 w2_ref, o_ref):
    flag = _mask_flag(m_ref, pl.program_id(0))
    xm = x_ref[...] * (1.0 - flag)
    h = _cat(h_ref)
    z = (jnp.dot(xm, w1_ref[...], preferred_element_type=jnp.float32,
                 precision=lax.Precision.HIGHEST)
         + flag * w1f_ref[...]
         + jnp.dot(h, w1h_ref[...], preferred_element_type=jnp.float32,
                 precision=lax.Precision.HIGHEST))
    z = jnp.maximum(_ln(z, g_ref[...], b_ref[...]), 0.0)
    o = jnp.dot(z, w2_ref[...], preferred_element_type=jnp.float32,
                 precision=lax.Precision.HIGHEST)
    _split_out(o_ref, o)


def _gin_body(h_ref, a_ref, eps_ref, w1_ref, g_ref, b_ref, w2_ref, dir_ref,
              lg_ref, lb_ref, o_ref):
    h = _cat(h_ref)
    a = _cat(a_ref)
    u = (1.0 + eps_ref[0]) * h + a
    z = jnp.dot(u, w1_ref[...], preferred_element_type=jnp.float32,
                 precision=lax.Precision.HIGHEST)
    z = jnp.maximum(_ln(z, g_ref[...], b_ref[...]), 0.0)
    y = jnp.dot(z, w2_ref[...], preferred_element_type=jnp.float32,
                 precision=lax.Precision.HIGHEST)
    v = jnp.maximum(y + h + dir_ref[...], 0.0)
    v = _ln(v, lg_ref[...], lb_ref[...])
    _split_out(o_ref, v)


def _out_body(h_ref, w1_ref, g_ref, b_ref, w2_ref, o_ref):
    h = _cat(h_ref)
    z = jnp.dot(h, w1_ref[...], preferred_element_type=jnp.float32,
                 precision=lax.Precision.HIGHEST)
    z = jnp.maximum(_ln(z, g_ref[...], b_ref[...]), 0.0)
    o_ref[...] = jnp.dot(z, w2_ref[...], preferred_element_type=jnp.float32,
                 precision=lax.Precision.HIGHEST)


def _full(shape):
    return pl.BlockSpec(shape, lambda i: tuple(0 for _ in shape))


_HSPEC = pl.BlockSpec((Q, BT, QW), lambda i: (0, i, 0))
_SMEM = pl.BlockSpec(memory_space=pltpu.MemorySpace.SMEM)


def _enc_in(xp, maskp, w1p, w1f, g, b, w2):
    return pl.pallas_call(
        _enc_in_body,
        grid=(N // BT,),
        in_specs=[
            pl.BlockSpec((BT, XPAD), lambda i: (i, 0)),
            _full((1, MPAD)),
            _full((XPAD, BOTT)),
            _full((1, BOTT)),
            _full((1, BOTT)),
            _full((1, BOTT)),
            _full((BOTT, HID)),
        ],
        out_specs=_HSPEC,
        out_shape=jax.ShapeDtypeStruct((Q, N, QW), jnp.float32),
    )(xp, maskp, w1p, w1f, g, b, w2)


def _dec_in(xp, maskp, h4, w1p, w1f, w1h, g, b, w2):
    return pl.pallas_call(
        _dec_in_body,
        grid=(N // BT,),
        in_specs=[
            pl.BlockSpec((BT, XPAD), lambda i: (i, 0)),
            _full((1, MPAD)),
            _HSPEC,
            _full((XPAD, BOTT)),
            _full((1, BOTT)),
            _full((HID, BOTT)),
            _full((1, BOTT)),
            _full((1, BOTT)),
            _full((BOTT, HID)),
        ],
        out_specs=_HSPEC,
        out_shape=jax.ShapeDtypeStruct((Q, N, QW), jnp.float32),
    )(xp, maskp, h4, w1p, w1f, w1h, g, b, w2)


def _gin_post(h4, agg4, eps, mlp, dir_row, ln_g, ln_b):
    return pl.pallas_call(
        _gin_body,
        grid=(N // BT,),
        in_specs=[
            _HSPEC,
            pl.BlockSpec((Q, BT, QW), lambda i: (0, i, 0)),
            _SMEM,
            _full((HID, BOTT)),
            _full((1, BOTT)),
            _full((1, BOTT)),
            _full((BOTT, HID)),
            _full((1, HID)),
            _full((1, HID)),
            _full((1, HID)),
        ],
        out_specs=_HSPEC,
        out_shape=jax.ShapeDtypeStruct((Q, N, QW), jnp.float32),
    )(h4, agg4, jnp.reshape(eps, (1,)), mlp["W1"],
      jnp.reshape(mlp["g"], (1, BOTT)), jnp.reshape(mlp["b"], (1, BOTT)),
      mlp["W2"], dir_row, jnp.reshape(ln_g, (1, HID)),
      jnp.reshape(ln_b, (1, HID)))


def _out_mlp(h4, p):
    bott = p["W1"].shape[1]
    return pl.pallas_call(
        _out_body,
        grid=(N // BT,),
        in_specs=[
            _HSPEC,
            _full((HID, bott)),
            _full((1, bott)),
            _full((1, bott)),
            _full((bott, IN_DIM)),
        ],
        out_specs=pl.BlockSpec((BT, IN_DIM), lambda i: (i, 0)),
        out_shape=jax.ShapeDtypeStruct((N, IN_DIM), jnp.float32),
    )(h4, p["W1"], jnp.reshape(p["g"], (1, bott)),
      jnp.reshape(p["b"], (1, bott)), p["W2"])


def _edge_plan(gather, scatter):
    padlen = EPAD - gather.shape[0]
    gp = jnp.concatenate([gather, jnp.zeros((padlen,), jnp.int32)])
    sp = jnp.concatenate([scatter, jnp.full((padlen,), N, jnp.int32)])
    gb = gp.reshape(NS, 1, TCH, CHUNK)
    gidx = jnp.concatenate([gb + q * N for q in range(Q)], axis=1)
    gidx = gidx.transpose(1, 0, 2, 3).reshape(Q * NS, 2, HC, CHUNK)
    sidx = sp.reshape(NS, 2, HC, CHUNK)
    return gidx, sidx


def _downup(h4, lp, down_idx, up_idx):
    agg = _agg(h4.reshape(Q * N, QW), *down_idx).reshape(Q, NPAD, QW)
    h4 = _gin_post(h4, agg, lp["down_eps"], lp["down_mlp"], lp["dir_emb"][0:1],
                   lp["ln1_g"], lp["ln1_b"])
    agg = _agg(h4.reshape(Q * N, QW), *up_idx).reshape(Q, NPAD, QW)
    h4 = _gin_post(h4, agg, lp["up_eps"], lp["up_mlp"], lp["dir_emb"][1:2],
                   lp["ln2_g"], lp["ln2_b"])
    return h4


def kernel(x, edge_index, mask_idx, params):
    x = x.astype(jnp.float32)
    src = edge_index[0].astype(jnp.int32)
    dst = edge_index[1].astype(jnp.int32)
    mask_idx = mask_idx.astype(jnp.int32)

    xp = jnp.pad(x, ((0, 0), (0, XPAD - IN_DIM)))
    maskp = jnp.pad(mask_idx, (0, MPAD - mask_idx.shape[0]),
                    constant_values=-1).reshape(1, MPAD)

    down_idx = _edge_plan(src, dst)
    up_idx = _edge_plan(dst, src)

    p = params["enc_in"]
    w1x = jnp.pad(p["W1"][:IN_DIM], ((0, XPAD - IN_DIM), (0, 0)))
    h4 = _enc_in(xp, maskp, w1x, p["W1"][IN_DIM:IN_DIM + 1],
                 jnp.reshape(p["g"], (1, BOTT)), jnp.reshape(p["b"], (1, BOTT)),
                 p["W2"])

    for lp in params["enc_layers"]:
        h4 = _downup(h4, lp, down_idx, up_idx)

    p = params["dec_in"]
    w1x = jnp.pad(p["W1"][:IN_DIM], ((0, XPAD - IN_DIM), (0, 0)))
    dh4 = _dec_in(xp, maskp, h4, w1x, p["W1"][IN_DIM:IN_DIM + 1],
                  p["W1"][IN_DIM + 1:],
                  jnp.reshape(p["g"], (1, BOTT)), jnp.reshape(p["b"], (1, BOTT)),
                  p["W2"])

    for lp in params["dec_layers"]:
        dh4 = _downup(dh4, lp, down_idx, up_idx)

    return _out_mlp(dh4, params["out"])


# speedup vs baseline: 1.0892x; 1.0892x over previous
import functools

import jax
import jax.numpy as jnp
from jax import lax
from jax.experimental import pallas as pl
from jax.experimental.pallas import tpu as pltpu
from jax.experimental.pallas import tpu_sc as plsc

N = 10000
IN_DIM = 19
HID = 256
BOTT = 128
Q = 2
QW = HID // Q
NPAD = 10240
NS = 16
NC = 2
CHUNK = 128
TCH = 163840 // (16 * CHUNK)
HC = TCH // 2
PT = TCH * CHUNK
EPAD = PT * NS
MPAD = 1536
XPAD = 128
BT = 1000
LNEPS = 1e-5


def _build_agg():
    mesh = plsc.VectorSubcoreMesh(core_axis_name="c", subcore_axis_name="s",
                                  num_cores=NC, num_subcores=NS)

    @functools.partial(
        pl.kernel,
        out_type=jax.ShapeDtypeStruct((Q * NPAD, QW), jnp.float32),
        mesh=mesh,
        scratch_types=[
            pltpu.VMEM((HC, CHUNK), jnp.int32),
            pltpu.VMEM((HC, CHUNK), jnp.int32),
            pltpu.VMEM((CHUNK, QW), jnp.float32),
            pltpu.VMEM_SHARED((NPAD, QW), jnp.float32),
            pltpu.SemaphoreType.DMA,
        ],
    )
    def agg(h4, gidx, sidx, out, gv, sv, rows, acc, sem):
        cid = lax.axis_index("c")
        sid = lax.axis_index("s")
        stripe = NPAD // NS
        ZROWS = 128

        for p in range(Q // NC):
            qid = cid * (Q // NC) + p
            row = qid * NS + sid

            @pl.loop(0, ZROWS)
            def _zr(r):
                @pl.loop(0, QW // 16)
                def _zc(k):
                    rows[r, pl.ds(k * 16, 16)] = jnp.zeros((16,),
                                                           jnp.float32)

            @pl.loop(0, stripe // ZROWS)
            def _za(k):
                pltpu.sync_copy(rows.at[pl.ds(0, ZROWS)],
                                acc.at[pl.ds(sid * stripe + k * ZROWS,
                                             ZROWS)])

            plsc.subcore_barrier()

            for half in range(2):
                pltpu.sync_copy(gidx.at[row, half], gv)
                pltpu.sync_copy(sidx.at[sid, half], sv)

                @pl.loop(0, HC)
                def _edges(j):
                    pltpu.async_copy(h4.at[gv.at[j]], rows, sem).wait()
                    pltpu.sync_copy(rows, acc.at[sv.at[j]], add=True)

            plsc.subcore_barrier()
            pltpu.sync_copy(acc.at[pl.ds(sid * stripe, stripe)],
                            out.at[pl.ds(qid * NPAD + sid * stripe, stripe)])
            plsc.subcore_barrier()

    return agg


_agg_cache = []


def _agg(h4, gidx, sidx):
    if not _agg_cache:
        _agg_cache.append(_build_agg())
    return _agg_cache[0](h4, gidx, sidx)


def _ln(h, g, b):
    m = jnp.mean(h, axis=-1, keepdims=True)
    v = jnp.mean((h - m) ** 2, axis=-1, keepdims=True)
    return (h - m) / jnp.sqrt(v + LNEPS) * g + b


def _cat(ref):
    return jnp.concatenate([ref[i] for i in range(Q)], axis=1)


def _split_out(o_ref, v):
    for i in range(Q):
        o_ref[i] = v[:, i * QW:(i + 1) * QW]


def _mask_flag(m_ref, i):
    rows = lax.broadcasted_iota(jnp.int32, (BT, 1), 0) + i * BT
    flag = jnp.any(m_ref[0][None, :] == rows, axis=1, keepdims=True)
    return flag.astype(jnp.float32)


def _enc_in_body(x_ref, m_ref, w1_ref, w1f_ref, g_ref, b_ref, w2_ref, o_ref):
    flag = _mask_flag(m_ref, pl.program_id(0))
    xm = x_ref[...] * (1.0 - flag)
    z = jnp.dot(xm, w1_ref[...], preferred_element_type=jnp.float32)
    z = z + flag * w1f_ref[...]
    z = jnp.maximum(_ln(z, g_ref[...], b_ref[...]), 0.0)
    h = jnp.dot(z, w2_ref[...], preferred_element_type=jnp.float32)
    _split_out(o_ref, h)


def _dec_in_body(x_ref, m_ref, h_ref, w1_ref, w1f_ref, w1h_ref, g_ref, b_ref,
                 w2_ref, o_ref):
    flag = _mask_flag(m_ref, pl.program_id(0))
    xm = x_ref[...] * (1.0 - flag)
    h = _cat(h_ref)
    z = (jnp.dot(xm, w1_ref[...], preferred_element_type=jnp.float32)
         + flag * w1f_ref[...]
         + jnp.dot(h, w1h_ref[...], preferred_element_type=jnp.float32))
    z = jnp.maximum(_ln(z, g_ref[...], b_ref[...]), 0.0)
    o = jnp.dot(z, w2_ref[...], preferred_element_type=jnp.float32)
    _split_out(o_ref, o)


def _gin_body(h_ref, a_ref, eps_ref, w1_ref, g_ref, b_ref, w2_ref, dir_ref,
              lg_ref, lb_ref, o_ref):
    h = _cat(h_ref)
    a = _cat(a_ref)
    u = (1.0 + eps_ref[0]) * h + a
    z = jnp.dot(u, w1_ref[...], preferred_element_type=jnp.float32)
    z = jnp.maximum(_ln(z, g_ref[...], b_ref[...]), 0.0)
    y = jnp.dot(z, w2_ref[...], preferred_element_type=jnp.float32)
    v = jnp.maximum(y + h + dir_ref[...], 0.0)
    v = _ln(v, lg_ref[...], lb_ref[...])
    _split_out(o_ref, v)


def _out_body(h_ref, w1_ref, g_ref, b_ref, w2_ref, o_ref):
    h = _cat(h_ref)
    z = jnp.dot(h, w1_ref[...], preferred_element_type=jnp.float32)
    z = jnp.maximum(_ln(z, g_ref[...], b_ref[...]), 0.0)
    o_ref[...] = jnp.dot(z, w2_ref[...], preferred_element_type=jnp.float32)


def _full(shape):
    return pl.BlockSpec(shape, lambda i: tuple(0 for _ in shape))


_HSPEC = pl.BlockSpec((Q, BT, QW), lambda i: (0, i, 0))
_SMEM = pl.BlockSpec(memory_space=pltpu.MemorySpace.SMEM)


def _enc_in(xp, maskp, w1p, w1f, g, b, w2):
    return pl.pallas_call(
        _enc_in_body,
        grid=(N // BT,),
        in_specs=[
            pl.BlockSpec((BT, XPAD), lambda i: (i, 0)),
            _full((1, MPAD)),
            _full((XPAD, BOTT)),
            _full((1, BOTT)),
            _full((1, BOTT)),
            _full((1, BOTT)),
            _full((BOTT, HID)),
        ],
        out_specs=_HSPEC,
        out_shape=jax.ShapeDtypeStruct((Q, N, QW), jnp.float32),
    )(xp, maskp, w1p, w1f, g, b, w2)


def _dec_in(xp, maskp, h4, w1p, w1f, w1h, g, b, w2):
    return pl.pallas_call(
        _dec_in_body,
        grid=(N // BT,),
        in_specs=[
            pl.BlockSpec((BT, XPAD), lambda i: (i, 0)),
            _full((1, MPAD)),
            _HSPEC,
            _full((XPAD, BOTT)),
            _full((1, BOTT)),
            _full((HID, BOTT)),
            _full((1, BOTT)),
            _full((1, BOTT)),
            _full((BOTT, HID)),
        ],
        out_specs=_HSPEC,
        out_shape=jax.ShapeDtypeStruct((Q, N, QW), jnp.float32),
    )(xp, maskp, h4, w1p, w1f, w1h, g, b, w2)


def _gin_post(h4, agg4, eps, mlp, dir_row, ln_g, ln_b):
    return pl.pallas_call(
        _gin_body,
        grid=(N // BT,),
        in_specs=[
            _HSPEC,
            pl.BlockSpec((Q, BT, QW), lambda i: (0, i, 0)),
            _SMEM,
            _full((HID, BOTT)),
            _full((1, BOTT)),
            _full((1, BOTT)),
            _full((BOTT, HID)),
            _full((1, HID)),
            _full((1, HID)),
            _full((1, HID)),
        ],
        out_specs=_HSPEC,
        out_shape=jax.ShapeDtypeStruct((Q, N, QW), jnp.float32),
    )(h4, agg4, jnp.reshape(eps, (1,)), mlp["W1"],
      jnp.reshape(mlp["g"], (1, BOTT)), jnp.reshape(mlp["b"], (1, BOTT)),
      mlp["W2"], dir_row, jnp.reshape(ln_g, (1, HID)),
      jnp.reshape(ln_b, (1, HID)))


def _out_mlp(h4, p):
    bott = p["W1"].shape[1]
    return pl.pallas_call(
        _out_body,
        grid=(N // BT,),
        in_specs=[
            _HSPEC,
            _full((HID, bott)),
            _full((1, bott)),
            _full((1, bott)),
            _full((bott, IN_DIM)),
        ],
        out_specs=pl.BlockSpec((BT, IN_DIM), lambda i: (i, 0)),
        out_shape=jax.ShapeDtypeStruct((N, IN_DIM), jnp.float32),
    )(h4, p["W1"], jnp.reshape(p["g"], (1, bott)),
      jnp.reshape(p["b"], (1, bott)), p["W2"])


def _edge_plan(gather, scatter):
    padlen = EPAD - gather.shape[0]
    gp = jnp.concatenate([gather, jnp.zeros((padlen,), jnp.int32)])
    sp = jnp.concatenate([scatter, jnp.full((padlen,), N, jnp.int32)])
    gb = gp.reshape(NS, 1, TCH, CHUNK)
    gidx = jnp.concatenate([gb + q * N for q in range(Q)], axis=1)
    gidx = gidx.transpose(1, 0, 2, 3).reshape(Q * NS, 2, HC, CHUNK)
    sidx = sp.reshape(NS, 2, HC, CHUNK)
    return gidx, sidx


def _downup(h4, lp, down_idx, up_idx):
    agg = _agg(h4.reshape(Q * N, QW), *down_idx).reshape(Q, NPAD, QW)
    h4 = _gin_post(h4, agg, lp["down_eps"], lp["down_mlp"], lp["dir_emb"][0:1],
                   lp["ln1_g"], lp["ln1_b"])
    agg = _agg(h4.reshape(Q * N, QW), *up_idx).reshape(Q, NPAD, QW)
    h4 = _gin_post(h4, agg, lp["up_eps"], lp["up_mlp"], lp["dir_emb"][1:2],
                   lp["ln2_g"], lp["ln2_b"])
    return h4


def kernel(x, edge_index, mask_idx, params):
    x = x.astype(jnp.float32)
    src = edge_index[0].astype(jnp.int32)
    dst = edge_index[1].astype(jnp.int32)
    mask_idx = mask_idx.astype(jnp.int32)

    xp = jnp.pad(x, ((0, 0), (0, XPAD - IN_DIM)))
    maskp = jnp.pad(mask_idx, (0, MPAD - mask_idx.shape[0]),
                    constant_values=-1).reshape(1, MPAD)

    down_idx = _edge_plan(src, dst)
    up_idx = _edge_plan(dst, src)

    p = params["enc_in"]
    w1x = jnp.pad(p["W1"][:IN_DIM], ((0, XPAD - IN_DIM), (0, 0)))
    h4 = _enc_in(xp, maskp, w1x, p["W1"][IN_DIM:IN_DIM + 1],
                 jnp.reshape(p["g"], (1, BOTT)), jnp.reshape(p["b"], (1, BOTT)),
                 p["W2"])

    for lp in params["enc_layers"]:
        h4 = _downup(h4, lp, down_idx, up_idx)

    p = params["dec_in"]
    w1x = jnp.pad(p["W1"][:IN_DIM], ((0, XPAD - IN_DIM), (0, 0)))
    dh4 = _dec_in(xp, maskp, h4, w1x, p["W1"][IN_DIM:IN_DIM + 1],
                  p["W1"][IN_DIM + 1:],
                  jnp.reshape(p["g"], (1, BOTT)), jnp.reshape(p["b"], (1, BOTT)),
                  p["W2"])

    for lp in params["dec_layers"]:
        dh4 = _downup(dh4, lp, down_idx, up_idx)

    return _out_mlp(dh4, params["out"])


# speedup vs baseline: 1.3613x; 1.2498x over previous
import functools

import jax
import jax.numpy as jnp
from jax import lax
from jax.experimental import pallas as pl
from jax.experimental.pallas import tpu as pltpu
from jax.experimental.pallas import tpu_sc as plsc

N = 10000
IN_DIM = 19
HID = 256
BOTT = 128
Q = 2
QW = HID // Q
NPAD = 10240
NS = 16
NC = 2
CHUNK = 128
TCH = 79
PT = TCH * CHUNK
EPAD = PT * NS
MPAD = 1536
XPAD = 128
BT = 1000
LNEPS = 1e-5


def _build_agg():
    mesh = plsc.VectorSubcoreMesh(core_axis_name="c", subcore_axis_name="s",
                                  num_cores=NC, num_subcores=NS)

    @functools.partial(
        pl.kernel,
        out_type=jax.ShapeDtypeStruct((Q * NPAD, QW), jnp.float32),
        mesh=mesh,
        scratch_types=[
            pltpu.VMEM((TCH, CHUNK), jnp.int32),
            pltpu.VMEM((TCH, CHUNK), jnp.int32),
            pltpu.VMEM((CHUNK, QW), jnp.float32),
            pltpu.VMEM_SHARED((NPAD, QW), jnp.float32),
            pltpu.SemaphoreType.DMA,
        ],
    )
    def agg(h4, gidx, sidx, out, gv, sv, rows, acc, sem):
        cid = lax.axis_index("c")
        sid = lax.axis_index("s")
        stripe = NPAD // NS

        pltpu.sync_copy(sidx.at[sid], sv)

        for p in range(Q // NC):
            qid = cid * (Q // NC) + p
            pltpu.sync_copy(gidx.at[qid * NS + sid], gv)

            @pl.loop(0, CHUNK)
            def _zr(r):
                @pl.loop(0, QW // 16)
                def _zc(k):
                    rows[r, pl.ds(k * 16, 16)] = jnp.zeros((16,),
                                                           jnp.float32)

            @pl.loop(0, stripe // CHUNK)
            def _za(k):
                pltpu.sync_copy(rows, acc.at[pl.ds(sid * stripe + k * CHUNK,
                                                   CHUNK)])

            plsc.subcore_barrier()

            @pl.loop(0, TCH)
            def _edges(j):
                pltpu.async_copy(h4.at[gv.at[j]], rows, sem).wait()
                pltpu.sync_copy(rows, acc.at[sv.at[j]], add=True)

            plsc.subcore_barrier()
            pltpu.sync_copy(acc.at[pl.ds(sid * stripe, stripe)],
                            out.at[pl.ds(qid * NPAD + sid * stripe, stripe)])
            plsc.subcore_barrier()

    return agg


_agg_cache = []


def _agg(h4, gidx, sidx):
    if not _agg_cache:
        _agg_cache.append(_build_agg())
    return _agg_cache[0](h4, gidx, sidx)


def _ln(h, g, b):
    m = jnp.mean(h, axis=-1, keepdims=True)
    v = jnp.mean((h - m) ** 2, axis=-1, keepdims=True)
    return (h - m) / jnp.sqrt(v + LNEPS) * g + b


def _cat(ref):
    return jnp.concatenate([ref[i] for i in range(Q)], axis=1)


def _split_out(o_ref, v):
    for i in range(Q):
        o_ref[i] = v[:, i * QW:(i + 1) * QW]


def _mask_flag(m_ref, i):
    rows = lax.broadcasted_iota(jnp.int32, (BT, 1), 0) + i * BT
    flag = jnp.any(m_ref[0][None, :] == rows, axis=1, keepdims=True)
    return flag.astype(jnp.float32)


def _enc_in_body(x_ref, m_ref, w1_ref, w1f_ref, g_ref, b_ref, w2_ref, o_ref):
    flag = _mask_flag(m_ref, pl.program_id(0))
    xm = x_ref[...] * (1.0 - flag)
    z = jnp.dot(xm, w1_ref[...], preferred_element_type=jnp.float32)
    z = z + flag * w1f_ref[...]
    z = jnp.maximum(_ln(z, g_ref[...], b_ref[...]), 0.0)
    h = jnp.dot(z, w2_ref[...], preferred_element_type=jnp.float32)
    _split_out(o_ref, h)


def _dec_in_body(x_ref, m_ref, h_ref, w1_ref, w1f_ref, w1h_ref, g_ref, b_ref,
                 w2_ref, o_ref):
    flag = _mask_flag(m_ref, pl.program_id(0))
    xm = x_ref[...] * (1.0 - flag)
    h = _cat(h_ref)
    z = (jnp.dot(xm, w1_ref[...], preferred_element_type=jnp.float32)
         + flag * w1f_ref[...]
         + jnp.dot(h, w1h_ref[...], preferred_element_type=jnp.float32))
    z = jnp.maximum(_ln(z, g_ref[...], b_ref[...]), 0.0)
    o = jnp.dot(z, w2_ref[...], preferred_element_type=jnp.float32)
    _split_out(o_ref, o)


def _gin_body(h_ref, a_ref, eps_ref, w1_ref, g_ref, b_ref, w2_ref, dir_ref,
              lg_ref, lb_ref, o_ref):
    h = _cat(h_ref)
    a = _cat(a_ref)
    u = (1.0 + eps_ref[0]) * h + a
    z = jnp.dot(u, w1_ref[...], preferred_element_type=jnp.float32)
    z = jnp.maximum(_ln(z, g_ref[...], b_ref[...]), 0.0)
    y = jnp.dot(z, w2_ref[...], preferred_element_type=jnp.float32)
    v = jnp.maximum(y + h + dir_ref[...], 0.0)
    v = _ln(v, lg_ref[...], lb_ref[...])
    _split_out(o_ref, v)


def _out_body(h_ref, w1_ref, g_ref, b_ref, w2_ref, o_ref):
    h = _cat(h_ref)
    z = jnp.dot(h, w1_ref[...], preferred_element_type=jnp.float32)
    z = jnp.maximum(_ln(z, g_ref[...], b_ref[...]), 0.0)
    o_ref[...] = jnp.dot(z, w2_ref[...], preferred_element_type=jnp.float32)


def _full(shape):
    return pl.BlockSpec(shape, lambda i: tuple(0 for _ in shape))


_HSPEC = pl.BlockSpec((Q, BT, QW), lambda i: (0, i, 0))
_SMEM = pl.BlockSpec(memory_space=pltpu.MemorySpace.SMEM)


def _enc_in(xp, maskp, w1p, w1f, g, b, w2):
    return pl.pallas_call(
        _enc_in_body,
        grid=(N // BT,),
        in_specs=[
            pl.BlockSpec((BT, XPAD), lambda i: (i, 0)),
            _full((1, MPAD)),
            _full((XPAD, BOTT)),
            _full((1, BOTT)),
            _full((1, BOTT)),
            _full((1, BOTT)),
            _full((BOTT, HID)),
        ],
        out_specs=_HSPEC,
        out_shape=jax.ShapeDtypeStruct((Q, N, QW), jnp.float32),
    )(xp, maskp, w1p, w1f, g, b, w2)


def _dec_in(xp, maskp, h4, w1p, w1f, w1h, g, b, w2):
    return pl.pallas_call(
        _dec_in_body,
        grid=(N // BT,),
        in_specs=[
            pl.BlockSpec((BT, XPAD), lambda i: (i, 0)),
            _full((1, MPAD)),
            _HSPEC,
            _full((XPAD, BOTT)),
            _full((1, BOTT)),
            _full((HID, BOTT)),
            _full((1, BOTT)),
            _full((1, BOTT)),
            _full((BOTT, HID)),
        ],
        out_specs=_HSPEC,
        out_shape=jax.ShapeDtypeStruct((Q, N, QW), jnp.float32),
    )(xp, maskp, h4, w1p, w1f, w1h, g, b, w2)


def _gin_post(h4, agg4, eps, mlp, dir_row, ln_g, ln_b):
    return pl.pallas_call(
        _gin_body,
        grid=(N // BT,),
        in_specs=[
            _HSPEC,
            pl.BlockSpec((Q, BT, QW), lambda i: (0, i, 0)),
            _SMEM,
            _full((HID, BOTT)),
            _full((1, BOTT)),
            _full((1, BOTT)),
            _full((BOTT, HID)),
            _full((1, HID)),
            _full((1, HID)),
            _full((1, HID)),
        ],
        out_specs=_HSPEC,
        out_shape=jax.ShapeDtypeStruct((Q, N, QW), jnp.float32),
    )(h4, agg4, jnp.reshape(eps, (1,)), mlp["W1"],
      jnp.reshape(mlp["g"], (1, BOTT)), jnp.reshape(mlp["b"], (1, BOTT)),
      mlp["W2"], dir_row, jnp.reshape(ln_g, (1, HID)),
      jnp.reshape(ln_b, (1, HID)))


def _out_mlp(h4, p):
    bott = p["W1"].shape[1]
    return pl.pallas_call(
        _out_body,
        grid=(N // BT,),
        in_specs=[
            _HSPEC,
            _full((HID, bott)),
            _full((1, bott)),
            _full((1, bott)),
            _full((bott, IN_DIM)),
        ],
        out_specs=pl.BlockSpec((BT, IN_DIM), lambda i: (i, 0)),
        out_shape=jax.ShapeDtypeStruct((N, IN_DIM), jnp.float32),
    )(h4, p["W1"], jnp.reshape(p["g"], (1, bott)),
      jnp.reshape(p["b"], (1, bott)), p["W2"])


def _edge_plan(gather, scatter):
    padlen = EPAD - gather.shape[0]
    gp = jnp.concatenate([gather, jnp.zeros((padlen,), jnp.int32)])
    sp = jnp.concatenate([scatter, jnp.full((padlen,), N, jnp.int32)])
    gb = gp.reshape(NS, TCH, CHUNK)
    gidx = jnp.concatenate([gb + q * N for q in range(Q)], axis=0)
    sidx = sp.reshape(NS, TCH, CHUNK)
    return gidx, sidx


def _downup(h4, lp, down_idx, up_idx):
    agg = _agg(h4.reshape(Q * N, QW), *down_idx).reshape(Q, NPAD, QW)
    h4 = _gin_post(h4, agg, lp["down_eps"], lp["down_mlp"], lp["dir_emb"][0:1],
                   lp["ln1_g"], lp["ln1_b"])
    agg = _agg(h4.reshape(Q * N, QW), *up_idx).reshape(Q, NPAD, QW)
    h4 = _gin_post(h4, agg, lp["up_eps"], lp["up_mlp"], lp["dir_emb"][1:2],
                   lp["ln2_g"], lp["ln2_b"])
    return h4


def kernel(x, edge_index, mask_idx, params):
    x = x.astype(jnp.float32)
    src = edge_index[0].astype(jnp.int32)
    dst = edge_index[1].astype(jnp.int32)
    mask_idx = mask_idx.astype(jnp.int32)

    xp = jnp.pad(x, ((0, 0), (0, XPAD - IN_DIM)))
    maskp = jnp.pad(mask_idx, (0, MPAD - mask_idx.shape[0]),
                    constant_values=-1).reshape(1, MPAD)

    down_idx = _edge_plan(src, dst)
    up_idx = _edge_plan(dst, src)

    p = params["enc_in"]
    w1x = jnp.pad(p["W1"][:IN_DIM], ((0, XPAD - IN_DIM), (0, 0)))
    h4 = _enc_in(xp, maskp, w1x, p["W1"][IN_DIM:IN_DIM + 1],
                 jnp.reshape(p["g"], (1, BOTT)), jnp.reshape(p["b"], (1, BOTT)),
                 p["W2"])

    for lp in params["enc_layers"]:
        h4 = _downup(h4, lp, down_idx, up_idx)

    p = params["dec_in"]
    w1x = jnp.pad(p["W1"][:IN_DIM], ((0, XPAD - IN_DIM), (0, 0)))
    dh4 = _dec_in(xp, maskp, h4, w1x, p["W1"][IN_DIM:IN_DIM + 1],
                  p["W1"][IN_DIM + 1:],
                  jnp.reshape(p["g"], (1, BOTT)), jnp.reshape(p["b"], (1, BOTT)),
                  p["W2"])

    for lp in params["dec_layers"]:
        dh4 = _downup(dh4, lp, down_idx, up_idx)

    return _out_mlp(dh4, params["out"])
